# baseline (speedup 1.0000x reference)
import jax
import jax.numpy as jnp
from jax import lax
from jax.experimental import pallas as pl
from jax.experimental.pallas import tpu as pltpu, tpu_sc as plsc

_LANES = 16
_NC, _NS = 2, 16
_NW = _NC * _NS


def _pop_gather_body(freq_hbm, cand_hbm, out_hbm, idx_v, rows_v, sem):
    wid = lax.axis_index("s") * _NC + lax.axis_index("c")
    chunk = idx_v.shape[0]
    base = wid * chunk
    pltpu.sync_copy(cand_hbm.at[pl.ds(base, chunk)], idx_v)
    pltpu.async_copy(freq_hbm.at[idx_v], rows_v, sem).wait()
    pltpu.sync_copy(rows_v, out_hbm.at[pl.ds(base, chunk)])


def kernel(tokens, candidates, item_freq):
    del tokens
    b, ncand = candidates.shape
    total = b * ncand
    vocab = item_freq.shape[-1]
    chunk = total // _NW
    assert total % (_NW * _LANES) == 0 and chunk % 8 == 0

    mesh = plsc.VectorSubcoreMesh(
        core_axis_name="c", subcore_axis_name="s",
        num_cores=_NC, num_subcores=_NS)
    run = pl.kernel(
        _pop_gather_body,
        out_type=jax.ShapeDtypeStruct((total,), jnp.float32),
        mesh=mesh,
        scratch_types=[
            pltpu.VMEM((chunk,), jnp.int32),
            pltpu.VMEM((chunk,), jnp.float32),
            pltpu.SemaphoreType.DMA,
        ],
        compiler_params=pltpu.CompilerParams(needs_layout_passes=False),
    )
    out = run(item_freq.reshape(vocab), candidates.reshape(total))
    out = out.reshape(b, ncand)
    return (out, out)
